# baseline (speedup 1.0000x reference)
import functools

import jax
import jax.numpy as jnp
from jax import lax
from jax.experimental import pallas as pl
from jax.experimental.pallas import tpu as pltpu
from jax.experimental.pallas import tpu_sc as plsc

LANES = 16
F0 = 0.725


def _make_hamilton(W):
    r, i, j, k = jnp.split(W, 4, axis=1)
    r2 = jnp.concatenate([r, -i, -j, -k], axis=0)
    i2 = jnp.concatenate([i, r, -k, j], axis=0)
    j2 = jnp.concatenate([j, k, r, -i], axis=0)
    k2 = jnp.concatenate([k, -j, i, r], axis=0)
    return jnp.concatenate([r2, i2, j2, k2], axis=1)


def _matmul_block(x_ref, h_ref, o_ref):
    o_ref[...] = jnp.dot(x_ref[...], h_ref[...],
                         preferred_element_type=jnp.float32)


def _combine_block(p_ref, o_ref):
    o_ref[...] = jnp.maximum(p_ref[0] + p_ref[1], 0.0)


def _pick_div(n, candidates):
    for c in candidates:
        if n % c == 0:
            return c
    return None


def _sc_spmm(pre_sup, packed, w, n_nodes):
    E = packed.shape[0]
    D = pre_sup.shape[1]
    info = plsc.get_sparse_core_info()
    NC, NS = info.num_cores, info.num_subcores

    K = 128
    total_chunks = E // K
    C0 = 2 * int(round(total_chunks * F0 / (2 * NS)))
    C1 = total_chunks // NS - C0

    ZB = _pick_div(n_nodes, (40, 80, 128, 200, 16, 8))
    n_blocks = n_nodes // ZB
    blocks_per_tile = -(-n_blocks // NS)

    mesh = plsc.VectorSubcoreMesh(core_axis_name="c", subcore_axis_name="s")

    @functools.partial(
        pl.kernel, mesh=mesh,
        out_type=jax.ShapeDtypeStruct((NC, n_nodes, D), jnp.float32),
        scratch_types=[
            [pltpu.VMEM((K,), jnp.int32)] * 2,
            [pltpu.VMEM((K,), jnp.int32)] * 2,
            [pltpu.VMEM((K,), jnp.int32)] * 2,
            [pltpu.VMEM((K,), jnp.float32)] * 2,
            [pltpu.VMEM((K, D), jnp.float32)] * 2,
            pltpu.VMEM((ZB, D), jnp.float32),
            pltpu.VMEM_SHARED((n_nodes, D), jnp.float32),
            [pltpu.SemaphoreType.DMA] * 2,
            [pltpu.SemaphoreType.DMA] * 2,
        ],
    )
    def spmm(pre_hbm, pk_hbm, w_hbm, out_hbm,
             pk_v, idx_v, dst_v, w_v, rows_v, zb_v, acc, gsem, ssem):
        c = lax.axis_index("c")
        s = lax.axis_index("s")
        nc_tile = jnp.where(c == 0, C0, C1)
        chunk0 = jnp.where(c == 0, s * C0, NS * C0 + s * C1)
        base0 = chunk0 * K

        def zero_row(i, _):
            for j in range(D // LANES):
                zb_v[i, pl.ds(j * LANES, LANES)] = jnp.zeros(
                    (LANES,), jnp.float32)
            return 0
        lax.fori_loop(0, ZB, zero_row, 0)
        for q in range(blocks_per_tile):
            blk = s + q * NS
            @pl.when(blk < n_blocks)
            def _():
                pltpu.sync_copy(zb_v, acc.at[pl.ds(blk * ZB, ZB)])
        plsc.subcore_barrier()

        def stage(t, b):
            base = base0 + t * K
            pltpu.make_async_copy(
                pk_hbm.at[pl.ds(base, K)], pk_v[b], ssem[b]).start()
            pltpu.make_async_copy(
                w_hbm.at[pl.ds(base, K)], w_v[b], ssem[b]).start()

        def fire(t, b):
            base = base0 + t * K
            pltpu.make_async_copy(
                pk_hbm.at[pl.ds(base, K)], pk_v[b], ssem[b]).wait()
            pltpu.make_async_copy(
                w_hbm.at[pl.ds(base, K)], w_v[b], ssem[b]).wait()

            def unpack_grp(g, _):
                sl = pl.ds(g * LANES, LANES)
                p = pk_v[b][sl]
                idx_v[b][sl] = lax.shift_right_logical(p, 16)
                dst_v[b][sl] = lax.bitwise_and(p, 0xFFFF)
                return 0
            lax.fori_loop(0, K // LANES, unpack_grp, 0)
            pltpu.make_async_copy(
                pre_hbm.at[idx_v[b]], rows_v[b], gsem[b]).start()

        def finish(b):
            pltpu.make_async_copy(
                pre_hbm.at[idx_v[b]], rows_v[b], gsem[b]).wait()

            def scale_grp(g, _):
                wv = w_v[b][pl.ds(g * LANES, LANES)]
                for l in range(LANES):
                    wgt = wv[l]
                    i = g * LANES + l
                    for j in range(D // LANES):
                        sl = pl.ds(j * LANES, LANES)
                        rows_v[b][i, sl] = rows_v[b][i, sl] * wgt
                return 0
            lax.fori_loop(0, K // LANES, scale_grp, 0)

            pltpu.sync_copy(rows_v[b], acc.at[dst_v[b]], add=True)

        @pl.when(nc_tile > 0)
        def _():
            stage(0, 0)
            fire(0, 0)
            stage(1, 1)

            def pair(p, _):
                t0 = 2 * p
                fire(t0 + 1, 1)
                finish(0)

                @pl.when(t0 + 2 < nc_tile)
                def _():
                    stage(t0 + 2, 0)
                    fire(t0 + 2, 0)
                finish(1)

                @pl.when(t0 + 3 < nc_tile)
                def _():
                    stage(t0 + 3, 1)
                return 0
            lax.fori_loop(0, nc_tile // 2, pair, 0)

        plsc.subcore_barrier()

        for q in range(blocks_per_tile):
            blk = s + q * NS
            @pl.when(blk < n_blocks)
            def _():
                r0 = blk * ZB
                pltpu.sync_copy(acc.at[pl.ds(r0, ZB)], zb_v)
                pltpu.sync_copy(zb_v, out_hbm.at[c, pl.ds(r0, ZB)])

    return spmm(pre_sup, packed, w)


def kernel(x, edge_index, edge_weight, W):
    N, D_in = x.shape
    hamilton = _make_hamilton(W)
    D = hamilton.shape[1]
    src = edge_index[0].astype(jnp.int32)
    dst = edge_index[1].astype(jnp.int32)
    packed = jnp.bitwise_or(jnp.left_shift(src, 16), dst)
    w = edge_weight.astype(jnp.float32)

    E = packed.shape[0]
    grain = 32 * 2 * 128
    E_pad = -(-E // grain) * grain
    if E_pad != E:
        packed = jnp.pad(packed, (0, E_pad - E))
        w = jnp.pad(w, (0, E_pad - E))

    BN = _pick_div(N, (1000, 500, 400, 250, 200, 100, 8))
    pre_sup = pl.pallas_call(
        _matmul_block,
        grid=(N // BN,),
        in_specs=[
            pl.BlockSpec((BN, D_in), lambda i: (i, 0)),
            pl.BlockSpec((D_in, D), lambda i: (0, 0)),
        ],
        out_specs=pl.BlockSpec((BN, D), lambda i: (i, 0)),
        out_shape=jax.ShapeDtypeStruct((N, D), jnp.float32),
    )(x, hamilton)

    partials = _sc_spmm(pre_sup, packed, w, N)

    out = pl.pallas_call(
        _combine_block,
        grid=(N // BN,),
        in_specs=[pl.BlockSpec((2, BN, D), lambda i: (0, i, 0))],
        out_specs=pl.BlockSpec((BN, D), lambda i: (i, 0)),
        out_shape=jax.ShapeDtypeStruct((N, D), jnp.float32),
    )(partials)
    return out


# speedup vs baseline: 1.0087x; 1.0087x over previous
import functools

import jax
import jax.numpy as jnp
from jax import lax
from jax.experimental import pallas as pl
from jax.experimental.pallas import tpu as pltpu
from jax.experimental.pallas import tpu_sc as plsc

LANES = 16
F0 = 0.8125


def _make_hamilton(W):
    r, i, j, k = jnp.split(W, 4, axis=1)
    r2 = jnp.concatenate([r, -i, -j, -k], axis=0)
    i2 = jnp.concatenate([i, r, -k, j], axis=0)
    j2 = jnp.concatenate([j, k, r, -i], axis=0)
    k2 = jnp.concatenate([k, -j, i, r], axis=0)
    return jnp.concatenate([r2, i2, j2, k2], axis=1)


def _matmul_block(x_ref, h_ref, o_ref):
    o_ref[...] = jnp.dot(x_ref[...], h_ref[...],
                         preferred_element_type=jnp.float32)


def _combine_block(p_ref, o_ref):
    o_ref[...] = jnp.maximum(p_ref[0] + p_ref[1], 0.0)


def _pick_div(n, candidates):
    for c in candidates:
        if n % c == 0:
            return c
    return None


def _sc_spmm(pre_sup, packed, w, n_nodes):
    E = packed.shape[0]
    D = pre_sup.shape[1]
    info = plsc.get_sparse_core_info()
    NC, NS = info.num_cores, info.num_subcores

    K = 128
    total_chunks = E // K
    C0 = 2 * int(round(total_chunks * F0 / (2 * NS)))
    C1 = total_chunks // NS - C0

    ZB = _pick_div(n_nodes, (40, 80, 128, 200, 16, 8))
    n_blocks = n_nodes // ZB
    blocks_per_tile = -(-n_blocks // NS)

    mesh = plsc.VectorSubcoreMesh(core_axis_name="c", subcore_axis_name="s")

    @functools.partial(
        pl.kernel, mesh=mesh,
        out_type=jax.ShapeDtypeStruct((NC, n_nodes, D), jnp.float32),
        scratch_types=[
            [pltpu.VMEM((K,), jnp.int32)] * 2,
            [pltpu.VMEM((K,), jnp.int32)] * 2,
            [pltpu.VMEM((K,), jnp.int32)] * 2,
            [pltpu.VMEM((K,), jnp.float32)] * 2,
            [pltpu.VMEM((K, D), jnp.float32)] * 2,
            pltpu.VMEM((ZB, D), jnp.float32),
            pltpu.VMEM_SHARED((n_nodes, D), jnp.float32),
            [pltpu.SemaphoreType.DMA] * 2,
            [pltpu.SemaphoreType.DMA] * 2,
        ],
    )
    def spmm(pre_hbm, pk_hbm, w_hbm, out_hbm,
             pk_v, idx_v, dst_v, w_v, rows_v, zb_v, acc, gsem, ssem):
        c = lax.axis_index("c")
        s = lax.axis_index("s")
        nc_tile = jnp.where(c == 0, C0, C1)
        chunk0 = jnp.where(c == 0, s * C0, NS * C0 + s * C1)
        base0 = chunk0 * K

        def zero_row(i, _):
            for j in range(D // LANES):
                zb_v[i, pl.ds(j * LANES, LANES)] = jnp.zeros(
                    (LANES,), jnp.float32)
            return 0
        lax.fori_loop(0, ZB, zero_row, 0)
        for q in range(blocks_per_tile):
            blk = s + q * NS
            @pl.when(blk < n_blocks)
            def _():
                pltpu.sync_copy(zb_v, acc.at[pl.ds(blk * ZB, ZB)])
        plsc.subcore_barrier()

        def stage(t, b):
            base = base0 + t * K
            pltpu.make_async_copy(
                pk_hbm.at[pl.ds(base, K)], pk_v[b], ssem[b]).start()
            pltpu.make_async_copy(
                w_hbm.at[pl.ds(base, K)], w_v[b], ssem[b]).start()

        def fire(t, b):
            base = base0 + t * K
            pltpu.make_async_copy(
                pk_hbm.at[pl.ds(base, K)], pk_v[b], ssem[b]).wait()
            pltpu.make_async_copy(
                w_hbm.at[pl.ds(base, K)], w_v[b], ssem[b]).wait()

            def unpack_grp(g, _):
                sl = pl.ds(g * LANES, LANES)
                p = pk_v[b][sl]
                idx_v[b][sl] = lax.shift_right_logical(p, 16)
                dst_v[b][sl] = lax.bitwise_and(p, 0xFFFF)
                return 0
            lax.fori_loop(0, K // LANES, unpack_grp, 0)
            pltpu.make_async_copy(
                pre_hbm.at[idx_v[b]], rows_v[b], gsem[b]).start()

        def finish(b):
            pltpu.make_async_copy(
                pre_hbm.at[idx_v[b]], rows_v[b], gsem[b]).wait()

            def scale_grp(g, _):
                wv = w_v[b][pl.ds(g * LANES, LANES)]
                for l in range(LANES):
                    wgt = wv[l]
                    i = g * LANES + l
                    for j in range(D // LANES):
                        sl = pl.ds(j * LANES, LANES)
                        rows_v[b][i, sl] = rows_v[b][i, sl] * wgt
                return 0
            lax.fori_loop(0, K // LANES, scale_grp, 0)

            pltpu.sync_copy(rows_v[b], acc.at[dst_v[b]], add=True)

        @pl.when(nc_tile > 0)
        def _():
            stage(0, 0)
            fire(0, 0)
            stage(1, 1)

            def pair(p, _):
                t0 = 2 * p
                fire(t0 + 1, 1)
                finish(0)

                @pl.when(t0 + 2 < nc_tile)
                def _():
                    stage(t0 + 2, 0)
                    fire(t0 + 2, 0)
                finish(1)

                @pl.when(t0 + 3 < nc_tile)
                def _():
                    stage(t0 + 3, 1)
                return 0
            lax.fori_loop(0, nc_tile // 2, pair, 0)

        plsc.subcore_barrier()

        for q in range(blocks_per_tile):
            blk = s + q * NS
            @pl.when(blk < n_blocks)
            def _():
                r0 = blk * ZB
                pltpu.sync_copy(acc.at[pl.ds(r0, ZB)], zb_v)
                pltpu.sync_copy(zb_v, out_hbm.at[c, pl.ds(r0, ZB)])

    return spmm(pre_sup, packed, w)


def kernel(x, edge_index, edge_weight, W):
    N, D_in = x.shape
    hamilton = _make_hamilton(W)
    D = hamilton.shape[1]
    src = edge_index[0].astype(jnp.int32)
    dst = edge_index[1].astype(jnp.int32)
    packed = jnp.bitwise_or(jnp.left_shift(src, 16), dst)
    w = edge_weight.astype(jnp.float32)

    E = packed.shape[0]
    grain = 32 * 2 * 128
    E_pad = -(-E // grain) * grain
    if E_pad != E:
        packed = jnp.pad(packed, (0, E_pad - E))
        w = jnp.pad(w, (0, E_pad - E))

    BN = _pick_div(N, (1000, 500, 400, 250, 200, 100, 8))
    pre_sup = pl.pallas_call(
        _matmul_block,
        grid=(N // BN,),
        in_specs=[
            pl.BlockSpec((BN, D_in), lambda i: (i, 0)),
            pl.BlockSpec((D_in, D), lambda i: (0, 0)),
        ],
        out_specs=pl.BlockSpec((BN, D), lambda i: (i, 0)),
        out_shape=jax.ShapeDtypeStruct((N, D), jnp.float32),
    )(x, hamilton)

    partials = _sc_spmm(pre_sup, packed, w, N)

    out = pl.pallas_call(
        _combine_block,
        grid=(N // BN,),
        in_specs=[pl.BlockSpec((2, BN, D), lambda i: (0, i, 0))],
        out_specs=pl.BlockSpec((BN, D), lambda i: (i, 0)),
        out_shape=jax.ShapeDtypeStruct((N, D), jnp.float32),
    )(partials)
    return out


# speedup vs baseline: 2.2870x; 2.2673x over previous
import functools

import jax
import jax.numpy as jnp
from jax import lax
from jax.experimental import pallas as pl
from jax.experimental.pallas import tpu as pltpu
from jax.experimental.pallas import tpu_sc as plsc

LANES = 16
F0 = 0.5


def _make_hamilton(W):
    r, i, j, k = jnp.split(W, 4, axis=1)
    r2 = jnp.concatenate([r, -i, -j, -k], axis=0)
    i2 = jnp.concatenate([i, r, -k, j], axis=0)
    j2 = jnp.concatenate([j, k, r, -i], axis=0)
    k2 = jnp.concatenate([k, -j, i, r], axis=0)
    return jnp.concatenate([r2, i2, j2, k2], axis=1)


def _matmul_block(x_ref, h_ref, o_ref):
    o_ref[...] = jnp.dot(x_ref[...], h_ref[...],
                         preferred_element_type=jnp.float32)


def _combine_block(p_ref, o_ref):
    o_ref[...] = jnp.maximum(p_ref[0] + p_ref[1], 0.0)


def _pick_div(n, candidates):
    for c in candidates:
        if n % c == 0:
            return c
    return None


def _sc_spmm(pre_sup, packed, w, n_nodes):
    E = packed.shape[0]
    D = pre_sup.shape[1]
    info = plsc.get_sparse_core_info()
    NC, NS = info.num_cores, info.num_subcores

    K = 128
    total_chunks = E // K
    C0 = 2 * int(round(total_chunks * F0 / (2 * NS)))
    C1 = total_chunks // NS - C0

    ZB = _pick_div(n_nodes, (40, 80, 128, 200, 16, 8))
    n_blocks = n_nodes // ZB
    blocks_per_tile = -(-n_blocks // NS)

    mesh = plsc.VectorSubcoreMesh(core_axis_name="c", subcore_axis_name="s")

    @functools.partial(
        pl.kernel, mesh=mesh,
        out_type=jax.ShapeDtypeStruct((NC, n_nodes, D), jnp.float32),
        scratch_types=[
            [pltpu.VMEM((K,), jnp.int32)] * 2,
            [pltpu.VMEM((K,), jnp.int32)] * 2,
            [pltpu.VMEM((K,), jnp.int32)] * 2,
            [pltpu.VMEM((K,), jnp.float32)] * 2,
            [pltpu.VMEM((K, D), jnp.float32)] * 2,
            pltpu.VMEM((ZB, D), jnp.float32),
            pltpu.VMEM_SHARED((n_nodes, D), jnp.float32),
            [pltpu.SemaphoreType.DMA] * 2,
            [pltpu.SemaphoreType.DMA] * 2,
        ],
    )
    def spmm(pre_hbm, pk_hbm, w_hbm, out_hbm,
             pk_v, idx_v, dst_v, w_v, rows_v, zb_v, acc, gsem, ssem):
        c = lax.axis_index("c")
        s = lax.axis_index("s")
        nc_tile = jnp.where(c == 0, C0, C1)
        chunk0 = jnp.where(c == 0, s * C0, NS * C0 + s * C1)
        base0 = chunk0 * K

        def zero_row(i, _):
            for j in range(D // LANES):
                zb_v[i, pl.ds(j * LANES, LANES)] = jnp.zeros(
                    (LANES,), jnp.float32)
            return 0
        lax.fori_loop(0, ZB, zero_row, 0)
        for q in range(blocks_per_tile):
            blk = s + q * NS
            @pl.when(blk < n_blocks)
            def _():
                pltpu.sync_copy(zb_v, acc.at[pl.ds(blk * ZB, ZB)])
        plsc.subcore_barrier()

        def stage(t, b):
            base = base0 + t * K
            pltpu.make_async_copy(
                pk_hbm.at[pl.ds(base, K)], pk_v[b], ssem[b]).start()
            pltpu.make_async_copy(
                w_hbm.at[pl.ds(base, K)], w_v[b], ssem[b]).start()

        def fire(t, b):
            base = base0 + t * K
            pltpu.make_async_copy(
                pk_hbm.at[pl.ds(base, K)], pk_v[b], ssem[b]).wait()
            pltpu.make_async_copy(
                w_hbm.at[pl.ds(base, K)], w_v[b], ssem[b]).wait()

            def unpack_grp(g, _):
                sl = pl.ds(g * LANES, LANES)
                p = pk_v[b][sl]
                idx_v[b][sl] = lax.shift_right_logical(p, 16)
                dst_v[b][sl] = lax.bitwise_and(p, 0xFFFF)
                return 0
            lax.fori_loop(0, K // LANES, unpack_grp, 0)
            pltpu.make_async_copy(
                pre_hbm.at[idx_v[b]], rows_v[b], gsem[b]).start()

        def finish(b):
            pltpu.make_async_copy(
                pre_hbm.at[idx_v[b]], rows_v[b], gsem[b]).wait()

            def scale_grp(g, _):
                wv = w_v[b][pl.ds(g * LANES, LANES)]
                for l in range(LANES):
                    wgt = wv[l]
                    i = g * LANES + l
                    for j in range(D // LANES):
                        sl = pl.ds(j * LANES, LANES)
                        rows_v[b][i, sl] = rows_v[b][i, sl] * wgt
                return 0
            lax.fori_loop(0, K // LANES, scale_grp, 0)

            pltpu.sync_copy(rows_v[b], acc.at[dst_v[b]], add=True)

        @pl.when(nc_tile > 0)
        def _():
            stage(0, 0)
            fire(0, 0)
            stage(1, 1)

            def pair(p, _):
                t0 = 2 * p
                fire(t0 + 1, 1)
                finish(0)

                @pl.when(t0 + 2 < nc_tile)
                def _():
                    stage(t0 + 2, 0)
                    fire(t0 + 2, 0)
                finish(1)

                @pl.when(t0 + 3 < nc_tile)
                def _():
                    stage(t0 + 3, 1)
                return 0
            lax.fori_loop(0, nc_tile // 2, pair, 0)

        plsc.subcore_barrier()

        for q in range(blocks_per_tile):
            blk = s + q * NS
            @pl.when(blk < n_blocks)
            def _():
                r0 = blk * ZB
                pltpu.sync_copy(acc.at[pl.ds(r0, ZB)], zb_v)
                pltpu.sync_copy(zb_v, out_hbm.at[c, pl.ds(r0, ZB)])

    return spmm(pre_sup, packed, w)


def kernel(x, edge_index, edge_weight, W):
    N, D_in = x.shape
    hamilton = _make_hamilton(W)
    D = hamilton.shape[1]
    src = edge_index[0].astype(jnp.int32)
    dst = edge_index[1].astype(jnp.int32)
    packed = jnp.bitwise_or(jnp.left_shift(src, 16), dst)
    w = edge_weight.astype(jnp.float32)

    E = packed.shape[0]
    grain = 32 * 2 * 128
    E_pad = -(-E // grain) * grain
    if E_pad != E:
        pad_src = jnp.arange(E_pad - E, dtype=jnp.int32) % N
        packed = jnp.concatenate([packed, jnp.left_shift(pad_src, 16)])
        w = jnp.pad(w, (0, E_pad - E))

    BN = _pick_div(N, (1000, 500, 400, 250, 200, 100, 8))
    pre_sup = pl.pallas_call(
        _matmul_block,
        grid=(N // BN,),
        in_specs=[
            pl.BlockSpec((BN, D_in), lambda i: (i, 0)),
            pl.BlockSpec((D_in, D), lambda i: (0, 0)),
        ],
        out_specs=pl.BlockSpec((BN, D), lambda i: (i, 0)),
        out_shape=jax.ShapeDtypeStruct((N, D), jnp.float32),
    )(x, hamilton)

    partials = _sc_spmm(pre_sup, packed, w, N)

    out = pl.pallas_call(
        _combine_block,
        grid=(N // BN,),
        in_specs=[pl.BlockSpec((2, BN, D), lambda i: (0, i, 0))],
        out_specs=pl.BlockSpec((BN, D), lambda i: (i, 0)),
        out_shape=jax.ShapeDtypeStruct((N, D), jnp.float32),
    )(partials)
    return out


# speedup vs baseline: 2.6528x; 1.1600x over previous
import functools

import jax
import jax.numpy as jnp
from jax import lax
from jax.experimental import pallas as pl
from jax.experimental.pallas import tpu as pltpu
from jax.experimental.pallas import tpu_sc as plsc

LANES = 16
F0 = 0.5


def _make_hamilton(W):
    r, i, j, k = jnp.split(W, 4, axis=1)
    r2 = jnp.concatenate([r, -i, -j, -k], axis=0)
    i2 = jnp.concatenate([i, r, -k, j], axis=0)
    j2 = jnp.concatenate([j, k, r, -i], axis=0)
    k2 = jnp.concatenate([k, -j, i, r], axis=0)
    return jnp.concatenate([r2, i2, j2, k2], axis=1)


def _matmul_block(x_ref, h_ref, o_ref):
    o_ref[...] = jnp.dot(x_ref[...], h_ref[...],
                         preferred_element_type=jnp.float32)


def _combine_block(p_ref, o_ref):
    o_ref[...] = jnp.maximum(p_ref[0] + p_ref[1], 0.0)


def _pick_div(n, candidates):
    for c in candidates:
        if n % c == 0:
            return c
    return None


def _sc_spmm(pre_sup, packed, w, n_nodes):
    E = packed.shape[0]
    D = pre_sup.shape[1]
    info = plsc.get_sparse_core_info()
    NC, NS = info.num_cores, info.num_subcores

    K = 128
    total_chunks = E // K
    C0 = 2 * int(round(total_chunks * F0 / (2 * NS)))
    C1 = total_chunks // NS - C0

    ZB = _pick_div(n_nodes, (40, 80, 128, 200, 16, 8))
    n_blocks = n_nodes // ZB
    blocks_per_tile = -(-n_blocks // NS)

    mesh = plsc.VectorSubcoreMesh(core_axis_name="c", subcore_axis_name="s")

    @functools.partial(
        pl.kernel, mesh=mesh,
        out_type=jax.ShapeDtypeStruct((NC, n_nodes, D), jnp.float32),
        scratch_types=[
            [pltpu.VMEM((K,), jnp.int32)] * 2,
            [pltpu.VMEM((K,), jnp.int32)] * 2,
            [pltpu.VMEM((K,), jnp.int32)] * 2,
            [pltpu.VMEM((K,), jnp.float32)] * 2,
            [pltpu.VMEM((K, D), jnp.float32)] * 2,
            pltpu.VMEM((ZB, D), jnp.float32),
            pltpu.VMEM_SHARED((n_nodes, D), jnp.float32),
            [pltpu.SemaphoreType.DMA] * 2,
            [pltpu.SemaphoreType.DMA] * 2,
            [pltpu.SemaphoreType.DMA] * 2,
        ],
    )
    def spmm(pre_hbm, pk_hbm, w_hbm, out_hbm,
             pk_v, idx_v, dst_v, w_v, rows_v, zb_v, acc, gsem, ssem, csem):
        c = lax.axis_index("c")
        s = lax.axis_index("s")
        nc_tile = jnp.where(c == 0, C0, C1)
        chunk0 = jnp.where(c == 0, s * C0, NS * C0 + s * C1)
        base0 = chunk0 * K

        def zero_row(i, _):
            for j in range(D // LANES):
                zb_v[i, pl.ds(j * LANES, LANES)] = jnp.zeros(
                    (LANES,), jnp.float32)
            return 0
        lax.fori_loop(0, ZB, zero_row, 0)
        for q in range(blocks_per_tile):
            blk = s + q * NS
            @pl.when(blk < n_blocks)
            def _():
                pltpu.sync_copy(zb_v, acc.at[pl.ds(blk * ZB, ZB)])
        plsc.subcore_barrier()

        def stage(t, b):
            base = base0 + t * K
            pltpu.make_async_copy(
                pk_hbm.at[pl.ds(base, K)], pk_v[b], ssem[b]).start()
            pltpu.make_async_copy(
                w_hbm.at[pl.ds(base, K)], w_v[b], ssem[b]).start()

        def fire(t, b):
            base = base0 + t * K
            pltpu.make_async_copy(
                pk_hbm.at[pl.ds(base, K)], pk_v[b], ssem[b]).wait()
            pltpu.make_async_copy(
                w_hbm.at[pl.ds(base, K)], w_v[b], ssem[b]).wait()

            @pl.when(t >= 2)
            def _():
                pltpu.make_async_copy(
                    rows_v[b], acc.at[dst_v[b]], csem[b]).wait()

            def unpack_grp(g, _):
                sl = pl.ds(g * LANES, LANES)
                p = pk_v[b][sl]
                idx_v[b][sl] = lax.shift_right_logical(p, 16)
                dst_v[b][sl] = lax.bitwise_and(p, 0xFFFF)
                return 0
            lax.fori_loop(0, K // LANES, unpack_grp, 0)
            pltpu.make_async_copy(
                pre_hbm.at[idx_v[b]], rows_v[b], gsem[b]).start()

        def finish(b):
            pltpu.make_async_copy(
                pre_hbm.at[idx_v[b]], rows_v[b], gsem[b]).wait()

            def scale_grp(g, _):
                wv = w_v[b][pl.ds(g * LANES, LANES)]
                for l in range(LANES):
                    wgt = wv[l]
                    i = g * LANES + l
                    for j in range(D // LANES):
                        sl = pl.ds(j * LANES, LANES)
                        rows_v[b][i, sl] = rows_v[b][i, sl] * wgt
                return 0
            lax.fori_loop(0, K // LANES, scale_grp, 0)

            pltpu.make_async_copy(
                rows_v[b], acc.at[dst_v[b]], csem[b]).start(add=True)

        @pl.when(nc_tile > 0)
        def _():
            stage(0, 0)
            fire(0, 0)
            stage(1, 1)

            def pair(p, _):
                t0 = 2 * p
                fire(t0 + 1, 1)
                finish(0)

                @pl.when(t0 + 2 < nc_tile)
                def _():
                    stage(t0 + 2, 0)
                    fire(t0 + 2, 0)
                finish(1)

                @pl.when(t0 + 3 < nc_tile)
                def _():
                    stage(t0 + 3, 1)
                return 0
            lax.fori_loop(0, nc_tile // 2, pair, 0)

            for b in range(2):
                pltpu.make_async_copy(
                    rows_v[b], acc.at[dst_v[b]], csem[b]).wait()

        plsc.subcore_barrier()

        for q in range(blocks_per_tile):
            blk = s + q * NS
            @pl.when(blk < n_blocks)
            def _():
                r0 = blk * ZB
                pltpu.sync_copy(acc.at[pl.ds(r0, ZB)], zb_v)
                pltpu.sync_copy(zb_v, out_hbm.at[c, pl.ds(r0, ZB)])

    return spmm(pre_sup, packed, w)


def kernel(x, edge_index, edge_weight, W):
    N, D_in = x.shape
    hamilton = _make_hamilton(W)
    D = hamilton.shape[1]
    src = edge_index[0].astype(jnp.int32)
    dst = edge_index[1].astype(jnp.int32)
    packed = jnp.bitwise_or(jnp.left_shift(src, 16), dst)
    w = edge_weight.astype(jnp.float32)

    E = packed.shape[0]
    grain = 32 * 2 * 128
    E_pad = -(-E // grain) * grain
    if E_pad != E:
        pad_node = jnp.arange(E_pad - E, dtype=jnp.int32) % N
        packed = jnp.concatenate(
            [packed, jnp.bitwise_or(jnp.left_shift(pad_node, 16), pad_node)])
        w = jnp.pad(w, (0, E_pad - E))

    BN = _pick_div(N, (1000, 500, 400, 250, 200, 100, 8))
    pre_sup = pl.pallas_call(
        _matmul_block,
        grid=(N // BN,),
        in_specs=[
            pl.BlockSpec((BN, D_in), lambda i: (i, 0)),
            pl.BlockSpec((D_in, D), lambda i: (0, 0)),
        ],
        out_specs=pl.BlockSpec((BN, D), lambda i: (i, 0)),
        out_shape=jax.ShapeDtypeStruct((N, D), jnp.float32),
    )(x, hamilton)

    partials = _sc_spmm(pre_sup, packed, w, N)

    out = pl.pallas_call(
        _combine_block,
        grid=(N // BN,),
        in_specs=[pl.BlockSpec((2, BN, D), lambda i: (0, i, 0))],
        out_specs=pl.BlockSpec((BN, D), lambda i: (i, 0)),
        out_shape=jax.ShapeDtypeStruct((N, D), jnp.float32),
    )(partials)
    return out


# speedup vs baseline: 2.6868x; 1.0128x over previous
import functools

import jax
import jax.numpy as jnp
from jax import lax
from jax.experimental import pallas as pl
from jax.experimental.pallas import tpu as pltpu
from jax.experimental.pallas import tpu_sc as plsc

LANES = 16
F0 = 0.5


def _make_hamilton(W):
    r, i, j, k = jnp.split(W, 4, axis=1)
    r2 = jnp.concatenate([r, -i, -j, -k], axis=0)
    i2 = jnp.concatenate([i, r, -k, j], axis=0)
    j2 = jnp.concatenate([j, k, r, -i], axis=0)
    k2 = jnp.concatenate([k, -j, i, r], axis=0)
    return jnp.concatenate([r2, i2, j2, k2], axis=1)


def _matmul_block(x_ref, h_ref, o_ref):
    o_ref[...] = jnp.dot(x_ref[...], h_ref[...],
                         preferred_element_type=jnp.float32)


def _combine_block(p_ref, o_ref):
    o_ref[...] = jnp.maximum(p_ref[0] + p_ref[1], 0.0)


def _pick_div(n, candidates):
    for c in candidates:
        if n % c == 0:
            return c
    return None


def _sc_spmm(pre_sup, packed, w, n_nodes):
    E = packed.shape[0]
    D = pre_sup.shape[1]
    info = plsc.get_sparse_core_info()
    NC, NS = info.num_cores, info.num_subcores

    K = 128
    total_chunks = E // K
    C0 = 2 * int(round(total_chunks * F0 / (2 * NS)))
    C1 = total_chunks // NS - C0

    ZB = _pick_div(n_nodes, (40, 80, 128, 200, 16, 8))
    n_blocks = n_nodes // ZB
    blocks_per_tile = -(-n_blocks // NS)

    mesh = plsc.VectorSubcoreMesh(core_axis_name="c", subcore_axis_name="s")

    @functools.partial(
        pl.kernel, mesh=mesh,
        out_type=jax.ShapeDtypeStruct((NC, n_nodes, D), jnp.float32),
        scratch_types=[
            [pltpu.VMEM((K,), jnp.int32)] * 2,
            [pltpu.VMEM((K,), jnp.int32)] * 2,
            [pltpu.VMEM((K,), jnp.int32)] * 2,
            [pltpu.VMEM((K,), jnp.float32)] * 2,
            [pltpu.VMEM((K, D), jnp.float32)] * 2,
            pltpu.VMEM((ZB, D), jnp.float32),
            pltpu.VMEM_SHARED((n_nodes, D), jnp.float32),
            [pltpu.SemaphoreType.DMA] * 2,
            [pltpu.SemaphoreType.DMA] * 2,
            [pltpu.SemaphoreType.DMA] * 2,
        ],
    )
    def spmm(pre_hbm, pk_hbm, w_hbm, out_hbm,
             pk_v, idx_v, dst_v, w_v, rows_v, zb_v, acc, gsem, ssem, csem):
        c = lax.axis_index("c")
        s = lax.axis_index("s")
        nc_tile = jnp.where(c == 0, C0, C1)
        chunk0 = jnp.where(c == 0, s * C0, NS * C0 + s * C1)
        base0 = chunk0 * K

        def zero_row(i, _):
            for j in range(D // LANES):
                zb_v[i, pl.ds(j * LANES, LANES)] = jnp.zeros(
                    (LANES,), jnp.float32)
            return 0
        lax.fori_loop(0, ZB, zero_row, 0)
        for q in range(blocks_per_tile):
            blk = s + q * NS
            @pl.when(blk < n_blocks)
            def _():
                pltpu.make_async_copy(
                    zb_v, acc.at[pl.ds(blk * ZB, ZB)], gsem[0]).start()
        for q in range(blocks_per_tile):
            blk = s + q * NS
            @pl.when(blk < n_blocks)
            def _():
                pltpu.make_async_copy(
                    zb_v, acc.at[pl.ds(blk * ZB, ZB)], gsem[0]).wait()
        plsc.subcore_barrier()

        def stage(t, b):
            base = base0 + t * K
            pltpu.make_async_copy(
                pk_hbm.at[pl.ds(base, K)], pk_v[b], ssem[b]).start()
            pltpu.make_async_copy(
                w_hbm.at[pl.ds(base, K)], w_v[b], ssem[b]).start()

        def fire(t, b):
            base = base0 + t * K
            pltpu.make_async_copy(
                pk_hbm.at[pl.ds(base, K)], pk_v[b], ssem[b]).wait()
            pltpu.make_async_copy(
                w_hbm.at[pl.ds(base, K)], w_v[b], ssem[b]).wait()

            @pl.when(t >= 2)
            def _():
                pltpu.make_async_copy(
                    rows_v[b], acc.at[dst_v[b]], csem[b]).wait()

            def unpack_grp(g, _):
                sl = pl.ds(g * LANES, LANES)
                p = pk_v[b][sl]
                idx_v[b][sl] = lax.shift_right_logical(p, 16)
                dst_v[b][sl] = lax.bitwise_and(p, 0xFFFF)
                return 0
            lax.fori_loop(0, K // LANES, unpack_grp, 0)
            pltpu.make_async_copy(
                pre_hbm.at[idx_v[b]], rows_v[b], gsem[b]).start()

        def finish(b):
            pltpu.make_async_copy(
                pre_hbm.at[idx_v[b]], rows_v[b], gsem[b]).wait()

            def scale_grp(g, _):
                wv = w_v[b][pl.ds(g * LANES, LANES)]
                for l in range(LANES):
                    wgt = wv[l]
                    i = g * LANES + l
                    for j in range(D // LANES):
                        sl = pl.ds(j * LANES, LANES)
                        rows_v[b][i, sl] = rows_v[b][i, sl] * wgt
                return 0
            lax.fori_loop(0, K // LANES, scale_grp, 0)

            pltpu.make_async_copy(
                rows_v[b], acc.at[dst_v[b]], csem[b]).start(add=True)

        @pl.when(nc_tile > 0)
        def _():
            stage(0, 0)
            fire(0, 0)
            stage(1, 1)

            def pair(p, _):
                t0 = 2 * p
                fire(t0 + 1, 1)
                finish(0)

                @pl.when(t0 + 2 < nc_tile)
                def _():
                    stage(t0 + 2, 0)
                    fire(t0 + 2, 0)
                finish(1)

                @pl.when(t0 + 3 < nc_tile)
                def _():
                    stage(t0 + 3, 1)
                return 0
            lax.fori_loop(0, nc_tile // 2, pair, 0)

            for b in range(2):
                pltpu.make_async_copy(
                    rows_v[b], acc.at[dst_v[b]], csem[b]).wait()

        plsc.subcore_barrier()

        for q in range(blocks_per_tile):
            blk = s + q * NS
            @pl.when(blk < n_blocks)
            def _():
                r0 = blk * ZB
                pltpu.make_async_copy(
                    acc.at[pl.ds(r0, ZB)],
                    out_hbm.at[c, pl.ds(r0, ZB)], gsem[1]).start()
        for q in range(blocks_per_tile):
            blk = s + q * NS
            @pl.when(blk < n_blocks)
            def _():
                r0 = blk * ZB
                pltpu.make_async_copy(
                    acc.at[pl.ds(r0, ZB)],
                    out_hbm.at[c, pl.ds(r0, ZB)], gsem[1]).wait()

    return spmm(pre_sup, packed, w)


def kernel(x, edge_index, edge_weight, W):
    N, D_in = x.shape
    hamilton = _make_hamilton(W)
    D = hamilton.shape[1]
    src = edge_index[0].astype(jnp.int32)
    dst = edge_index[1].astype(jnp.int32)
    packed = jnp.bitwise_or(jnp.left_shift(src, 16), dst)
    w = edge_weight.astype(jnp.float32)

    E = packed.shape[0]
    grain = 32 * 2 * 128
    E_pad = -(-E // grain) * grain
    if E_pad != E:
        pad_node = jnp.arange(E_pad - E, dtype=jnp.int32) % N
        packed = jnp.concatenate(
            [packed, jnp.bitwise_or(jnp.left_shift(pad_node, 16), pad_node)])
        w = jnp.pad(w, (0, E_pad - E))

    BN = _pick_div(N, (1000, 500, 400, 250, 200, 100, 8))
    pre_sup = pl.pallas_call(
        _matmul_block,
        grid=(N // BN,),
        in_specs=[
            pl.BlockSpec((BN, D_in), lambda i: (i, 0)),
            pl.BlockSpec((D_in, D), lambda i: (0, 0)),
        ],
        out_specs=pl.BlockSpec((BN, D), lambda i: (i, 0)),
        out_shape=jax.ShapeDtypeStruct((N, D), jnp.float32),
    )(x, hamilton)

    partials = _sc_spmm(pre_sup, packed, w, N)

    out = pl.pallas_call(
        _combine_block,
        grid=(N // BN,),
        in_specs=[pl.BlockSpec((2, BN, D), lambda i: (0, i, 0))],
        out_specs=pl.BlockSpec((BN, D), lambda i: (i, 0)),
        out_shape=jax.ShapeDtypeStruct((N, D), jnp.float32),
    )(partials)
    return out


# speedup vs baseline: 2.7042x; 1.0065x over previous
import functools

import jax
import jax.numpy as jnp
from jax import lax
from jax.experimental import pallas as pl
from jax.experimental.pallas import tpu as pltpu
from jax.experimental.pallas import tpu_sc as plsc

LANES = 16
F0 = 0.5


def _make_hamilton(W):
    r, i, j, k = jnp.split(W, 4, axis=1)
    r2 = jnp.concatenate([r, -i, -j, -k], axis=0)
    i2 = jnp.concatenate([i, r, -k, j], axis=0)
    j2 = jnp.concatenate([j, k, r, -i], axis=0)
    k2 = jnp.concatenate([k, -j, i, r], axis=0)
    return jnp.concatenate([r2, i2, j2, k2], axis=1)


def _matmul_block(x_ref, h_ref, o_ref):
    o_ref[...] = jnp.dot(x_ref[...], h_ref[...],
                         preferred_element_type=jnp.float32)


def _combine_block(p_ref, o_ref):
    o_ref[...] = jnp.maximum(p_ref[0] + p_ref[1], 0.0)


def _pick_div(n, candidates):
    for c in candidates:
        if n % c == 0:
            return c
    return None


def _sc_spmm(pre_sup, packed, w, n_nodes):
    E = packed.shape[0]
    D = pre_sup.shape[1]
    info = plsc.get_sparse_core_info()
    NC, NS = info.num_cores, info.num_subcores

    K = 160
    total_chunks = E // K
    C0 = 2 * int(round(total_chunks * F0 / (2 * NS)))
    C1 = total_chunks // NS - C0

    ZB = _pick_div(n_nodes, (40, 80, 128, 200, 16, 8))
    n_blocks = n_nodes // ZB
    blocks_per_tile = -(-n_blocks // NS)

    mesh = plsc.VectorSubcoreMesh(core_axis_name="c", subcore_axis_name="s")

    @functools.partial(
        pl.kernel, mesh=mesh,
        out_type=jax.ShapeDtypeStruct((NC, n_nodes, D), jnp.float32),
        scratch_types=[
            [pltpu.VMEM((K,), jnp.int32)] * 2,
            [pltpu.VMEM((K,), jnp.int32)] * 2,
            [pltpu.VMEM((K,), jnp.int32)] * 2,
            [pltpu.VMEM((K,), jnp.float32)] * 2,
            [pltpu.VMEM((K, D), jnp.float32)] * 2,
            pltpu.VMEM((ZB, D), jnp.float32),
            pltpu.VMEM_SHARED((n_nodes, D), jnp.float32),
            [pltpu.SemaphoreType.DMA] * 2,
            [pltpu.SemaphoreType.DMA] * 2,
            [pltpu.SemaphoreType.DMA] * 2,
        ],
    )
    def spmm(pre_hbm, pk_hbm, w_hbm, out_hbm,
             pk_v, idx_v, dst_v, w_v, rows_v, zb_v, acc, gsem, ssem, csem):
        c = lax.axis_index("c")
        s = lax.axis_index("s")
        nc_tile = jnp.where(c == 0, C0, C1)
        chunk0 = jnp.where(c == 0, s * C0, NS * C0 + s * C1)
        base0 = chunk0 * K

        def zero_row(i, _):
            for j in range(D // LANES):
                zb_v[i, pl.ds(j * LANES, LANES)] = jnp.zeros(
                    (LANES,), jnp.float32)
            return 0
        lax.fori_loop(0, ZB, zero_row, 0)
        for q in range(blocks_per_tile):
            blk = s + q * NS
            @pl.when(blk < n_blocks)
            def _():
                pltpu.make_async_copy(
                    zb_v, acc.at[pl.ds(blk * ZB, ZB)], gsem[0]).start()
        for q in range(blocks_per_tile):
            blk = s + q * NS
            @pl.when(blk < n_blocks)
            def _():
                pltpu.make_async_copy(
                    zb_v, acc.at[pl.ds(blk * ZB, ZB)], gsem[0]).wait()
        plsc.subcore_barrier()

        def stage(t, b):
            base = base0 + t * K
            pltpu.make_async_copy(
                pk_hbm.at[pl.ds(base, K)], pk_v[b], ssem[b]).start()
            pltpu.make_async_copy(
                w_hbm.at[pl.ds(base, K)], w_v[b], ssem[b]).start()

        def fire(t, b):
            base = base0 + t * K
            pltpu.make_async_copy(
                pk_hbm.at[pl.ds(base, K)], pk_v[b], ssem[b]).wait()
            pltpu.make_async_copy(
                w_hbm.at[pl.ds(base, K)], w_v[b], ssem[b]).wait()

            @pl.when(t >= 2)
            def _():
                pltpu.make_async_copy(
                    rows_v[b], acc.at[dst_v[b]], csem[b]).wait()

            def unpack_grp(g, _):
                sl = pl.ds(g * LANES, LANES)
                p = pk_v[b][sl]
                idx_v[b][sl] = lax.shift_right_logical(p, 16)
                dst_v[b][sl] = lax.bitwise_and(p, 0xFFFF)
                return 0
            lax.fori_loop(0, K // LANES, unpack_grp, 0)
            pltpu.make_async_copy(
                pre_hbm.at[idx_v[b]], rows_v[b], gsem[b]).start()

        def finish(b):
            pltpu.make_async_copy(
                pre_hbm.at[idx_v[b]], rows_v[b], gsem[b]).wait()

            def scale_grp(g, _):
                wv = w_v[b][pl.ds(g * LANES, LANES)]
                for l in range(LANES):
                    wgt = wv[l]
                    i = g * LANES + l
                    for j in range(D // LANES):
                        sl = pl.ds(j * LANES, LANES)
                        rows_v[b][i, sl] = rows_v[b][i, sl] * wgt
                return 0
            lax.fori_loop(0, K // LANES, scale_grp, 0)

            pltpu.make_async_copy(
                rows_v[b], acc.at[dst_v[b]], csem[b]).start(add=True)

        @pl.when(nc_tile > 0)
        def _():
            stage(0, 0)
            fire(0, 0)
            stage(1, 1)

            def pair(p, _):
                t0 = 2 * p
                fire(t0 + 1, 1)
                finish(0)

                @pl.when(t0 + 2 < nc_tile)
                def _():
                    stage(t0 + 2, 0)
                    fire(t0 + 2, 0)
                finish(1)

                @pl.when(t0 + 3 < nc_tile)
                def _():
                    stage(t0 + 3, 1)
                return 0
            lax.fori_loop(0, nc_tile // 2, pair, 0)

            for b in range(2):
                pltpu.make_async_copy(
                    rows_v[b], acc.at[dst_v[b]], csem[b]).wait()

        plsc.subcore_barrier()

        for q in range(blocks_per_tile):
            blk = s + q * NS
            @pl.when(blk < n_blocks)
            def _():
                r0 = blk * ZB
                pltpu.make_async_copy(
                    acc.at[pl.ds(r0, ZB)],
                    out_hbm.at[c, pl.ds(r0, ZB)], gsem[1]).start()
        for q in range(blocks_per_tile):
            blk = s + q * NS
            @pl.when(blk < n_blocks)
            def _():
                r0 = blk * ZB
                pltpu.make_async_copy(
                    acc.at[pl.ds(r0, ZB)],
                    out_hbm.at[c, pl.ds(r0, ZB)], gsem[1]).wait()

    return spmm(pre_sup, packed, w)


def kernel(x, edge_index, edge_weight, W):
    N, D_in = x.shape
    hamilton = _make_hamilton(W)
    D = hamilton.shape[1]
    src = edge_index[0].astype(jnp.int32)
    dst = edge_index[1].astype(jnp.int32)
    packed = jnp.bitwise_or(jnp.left_shift(src, 16), dst)
    w = edge_weight.astype(jnp.float32)

    E = packed.shape[0]
    grain = 32 * 2 * 160
    E_pad = -(-E // grain) * grain
    if E_pad != E:
        pad_node = jnp.arange(E_pad - E, dtype=jnp.int32) % N
        packed = jnp.concatenate(
            [packed, jnp.bitwise_or(jnp.left_shift(pad_node, 16), pad_node)])
        w = jnp.pad(w, (0, E_pad - E))

    BN = _pick_div(N, (1000, 500, 400, 250, 200, 100, 8))
    pre_sup = pl.pallas_call(
        _matmul_block,
        grid=(N // BN,),
        in_specs=[
            pl.BlockSpec((BN, D_in), lambda i: (i, 0)),
            pl.BlockSpec((D_in, D), lambda i: (0, 0)),
        ],
        out_specs=pl.BlockSpec((BN, D), lambda i: (i, 0)),
        out_shape=jax.ShapeDtypeStruct((N, D), jnp.float32),
    )(x, hamilton)

    partials = _sc_spmm(pre_sup, packed, w, N)

    out = pl.pallas_call(
        _combine_block,
        grid=(N // BN,),
        in_specs=[pl.BlockSpec((2, BN, D), lambda i: (0, i, 0))],
        out_specs=pl.BlockSpec((BN, D), lambda i: (i, 0)),
        out_shape=jax.ShapeDtypeStruct((N, D), jnp.float32),
    )(partials)
    return out
